# baseline (speedup 1.0000x reference)
import functools

import jax
import jax.numpy as jnp
from jax import lax
from jax.experimental import pallas as pl
from jax.experimental.pallas import tpu as pltpu
from jax.experimental.pallas import tpu_sc as plsc

N = 10000
K = 16
C = 256
LANES = 16
PTS_W = 4

NW = 32
NP = 10240
PW = NP // NW
CH = 8
NCHUNK = PW // CH
LAST = NCHUNK - 1
CV = C // LANES


def _mm_body(x_ref, wt_ref, b_ref, o_ref):
    o_ref[...] = (
        jnp.dot(x_ref[...], wt_ref[...], preferred_element_type=jnp.float32)
        + b_ref[...]
    ).astype(jnp.bfloat16)


def _linear(feats_pad, Wt, b):
    grid = NP // 1024
    return pl.pallas_call(
        _mm_body,
        grid=(grid,),
        in_specs=[
            pl.BlockSpec((1024, C), lambda i: (i, 0)),
            pl.BlockSpec((C, C), lambda i: (0, 0)),
            pl.BlockSpec((1, C), lambda i: (0, 0)),
        ],
        out_specs=pl.BlockSpec((1024, C), lambda i: (i, 0)),
        out_shape=jax.ShapeDtypeStruct((NP, C), jnp.bfloat16),
    )(feats_pad, Wt, b)


def _tree_sum(vs):
    while len(vs) > 1:
        nxt = [vs[i] + vs[i + 1] for i in range(0, len(vs) - 1, 2)]
        if len(vs) % 2:
            nxt.append(vs[-1])
        vs = nxt
    return vs[0]


def _sc_attend_body(f_hbm, ni_hbm, pts_hbm, out_hbm,
                    idx_a, idx_b, nbrf_a, nbrf_b, q_a, q_b, out_a, out_b,
                    pts_v, accf_v, nidx_s,
                    sem_ni_a, sem_ni_b, sem_g_a, sem_g_b,
                    sem_q_a, sem_q_b, sem_o_a, sem_o_b):
    wid = lax.axis_index("s") * 2 + lax.axis_index("c")
    base = wid * PW
    lane_ids = lax.iota(jnp.int32, LANES)
    row_base = lane_ids * LANES

    bufs = [
        (idx_a, nbrf_a, q_a, out_a, sem_ni_a, sem_g_a, sem_q_a, sem_o_a),
        (idx_b, nbrf_b, q_b, out_b, sem_ni_b, sem_g_b, sem_q_b, sem_o_b),
    ]

    def ni_copy(c, idxr, sem):
        return pltpu.make_async_copy(
            ni_hbm.at[pl.ds((base + c * CH) * K, CH * K)], idxr, sem)

    def g_copy(idxr, nbr, sem):
        return pltpu.make_async_copy(f_hbm.at[idxr], nbr, sem)

    def q_copy(c, qr, sem):
        return pltpu.make_async_copy(
            f_hbm.at[pl.ds(base + c * CH, CH)], qr, sem)

    def o_copy(c, outr, sem):
        return pltpu.make_async_copy(
            outr, out_hbm.at[pl.ds(base + c * CH, CH)], sem)

    ni_copy(0, idx_a, sem_ni_a).start()
    ni_copy(1, idx_b, sem_ni_b).start()
    pltpu.sync_copy(pts_hbm, pts_v)
    ni_copy(0, idx_a, sem_ni_a).wait()
    g_copy(idx_a, nbrf_a, sem_g_a).start()
    q_copy(0, q_a, sem_q_a).start()

    def compute_chunk(nidx_s, nbrX, qX, outX):
        def point_body(p, _):
            qs = [plsc.bitcast(qX[p, pl.ds(c * LANES, LANES)], jnp.bfloat16)
                  for c in range(C // 32)]
            for k in range(K):
                accs = [None] * 4
                for c in range(C // 32):
                    pr = qs[c] * plsc.bitcast(
                        nbrX[p * K + k, pl.ds(c * LANES, LANES)], jnp.bfloat16)
                    u0, u1 = plsc.unpack(pr, format=plsc.PackFormat.INTERLEAVED)
                    i0 = 2 * (c % 2)
                    accs[i0] = u0 if accs[i0] is None else accs[i0] + u0
                    accs[i0 + 1] = u1 if accs[i0 + 1] is None else accs[i0 + 1] + u1
                accf_v[pl.ds(k * LANES, LANES)] = (
                    (accs[0] + accs[1]) + (accs[2] + accs[3]))
            dots = _tree_sum(
                [plsc.load_gather(accf_v, [row_base + l]) for l in range(LANES)])
            e = jnp.exp(dots * (1.0 / 16.0))
            nidx = nidx_s[pl.ds(p * K, LANES)] * PTS_W
            px = plsc.load_gather(pts_v, [nidx])
            py = plsc.load_gather(pts_v, [nidx + 1])
            pz = plsc.load_gather(pts_v, [nidx + 2])
            accf_v[pl.ds(0, LANES)] = e * px
            accf_v[pl.ds(LANES, LANES)] = e * py
            accf_v[pl.ds(2 * LANES, LANES)] = e * pz
            accf_v[pl.ds(3 * LANES, LANES)] = e
            t = _tree_sum(
                [plsc.load_gather(accf_v, [row_base + l]) for l in range(LANES)])
            outX[p, :] = t / t[3]
            return 0

        lax.fori_loop(0, CH, point_body, 0)

    def step_body(s, _):
        for b in range(2):
            (idxX, nbrX, qX, outX, sem_niX, sem_gX, sem_qX, sem_oX) = bufs[b]
            (idxY, nbrY, qY, outY, sem_niY, sem_gY, sem_qY, sem_oY) = bufs[1 - b]
            g = 2 * s + b
            g1 = jnp.minimum(g + 1, LAST)
            g2 = jnp.minimum(g + 2, LAST)
            ni_copy(g1, idxY, sem_niY).wait()
            g_copy(idxX, nbrX, sem_gX).wait()
            q_copy(g, qX, sem_qX).wait()
            for j in range(CH):
                nidx_s[pl.ds(j * LANES, LANES)] = idxX[pl.ds(j * LANES, LANES)]
            ni_copy(g2, idxX, sem_niX).start()
            g_copy(idxY, nbrY, sem_gY).start()
            q_copy(g1, qY, sem_qY).start()

            @pl.when(g >= 2)
            def _():
                o_copy(g - 2, outX, sem_oX).wait()

            compute_chunk(nidx_s, nbrX, qX, outX)
            o_copy(g, outX, sem_oX).start()
        return 0

    lax.fori_loop(0, NCHUNK // 2, step_body, 0)

    ni_copy(LAST, idx_b, sem_ni_b).wait()
    g_copy(idx_a, nbrf_a, sem_g_a).wait()
    q_copy(LAST, q_a, sem_q_a).wait()
    o_copy(LAST - 1, out_a, sem_o_a).wait()
    o_copy(LAST, out_b, sem_o_b).wait()


_sc_attend = functools.partial(
    pl.kernel,
    mesh=plsc.VectorSubcoreMesh(core_axis_name="c", subcore_axis_name="s"),
    compiler_params=pltpu.CompilerParams(needs_layout_passes=False),
    out_type=jax.ShapeDtypeStruct((NP, LANES), jnp.float32),
    scratch_types=[
        pltpu.VMEM((CH * K,), jnp.int32),
        pltpu.VMEM((CH * K,), jnp.int32),
        pltpu.VMEM((CH * K, C // 2), jnp.int32),
        pltpu.VMEM((CH * K, C // 2), jnp.int32),
        pltpu.VMEM((CH, C // 2), jnp.int32),
        pltpu.VMEM((CH, C // 2), jnp.int32),
        pltpu.VMEM((CH, LANES), jnp.float32),
        pltpu.VMEM((CH, LANES), jnp.float32),
        pltpu.VMEM((N * PTS_W,), jnp.float32),
        pltpu.VMEM((K * LANES,), jnp.float32),
        pltpu.VMEM((CH * K,), jnp.int32),
        pltpu.SemaphoreType.DMA,
        pltpu.SemaphoreType.DMA,
        pltpu.SemaphoreType.DMA,
        pltpu.SemaphoreType.DMA,
        pltpu.SemaphoreType.DMA,
        pltpu.SemaphoreType.DMA,
        pltpu.SemaphoreType.DMA,
        pltpu.SemaphoreType.DMA,
    ],
)(_sc_attend_body)


@jax.jit
def kernel(points, feats, neighbor_indices, W, b):
    ni = neighbor_indices.astype(jnp.int32)
    own = jnp.broadcast_to(jnp.arange(N, dtype=jnp.int32)[:, None], (N, K))
    ni = jnp.where(ni < N, ni, own)
    ni_flat = jnp.pad(ni.reshape(-1), (0, (NP - N) * K))

    feats_pad = jnp.pad(feats, ((0, NP - N), (0, 0)))
    f_bf = _linear(feats_pad, W.T, b[None, :])
    f = lax.bitcast_convert_type(
        f_bf.reshape(NP, C // 2, 2), jnp.int32)

    pts_pad = jnp.pad(points, ((0, 0), (0, PTS_W - 3))).reshape(-1)
    out = _sc_attend(f, ni_flat, pts_pad)
    return out[:N, :3]


# speedup vs baseline: 1.4159x; 1.4159x over previous
import functools

import jax
import jax.numpy as jnp
from jax import lax
from jax.experimental import pallas as pl
from jax.experimental.pallas import tpu as pltpu
from jax.experimental.pallas import tpu_sc as plsc

N = 10000
K = 16
C = 256
CW = C // 2
LANES = 16
PTS_W = 4

NW = 32
NP = 10240
PW = NP // NW
CH = 8
NCHUNK = PW // CH
LAST = NCHUNK - 1
NBUF = 4


def _mm_body(x_ref, wt_ref, b_ref, o_ref):
    acc = (
        jnp.dot(x_ref[...], wt_ref[...], preferred_element_type=jnp.float32)
        + b_ref[...]
    ).astype(jnp.bfloat16)
    lo = lax.bitcast_convert_type(acc[:, :CW], jnp.uint16).astype(jnp.uint32)
    hi = lax.bitcast_convert_type(acc[:, CW:], jnp.uint16).astype(jnp.uint32)
    o_ref[...] = lax.bitcast_convert_type(lo | (hi << 16), jnp.int32)


def _linear(feats_pad, Wt, b):
    grid = NP // 1024
    return pl.pallas_call(
        _mm_body,
        grid=(grid,),
        in_specs=[
            pl.BlockSpec((1024, C), lambda i: (i, 0)),
            pl.BlockSpec((C, C), lambda i: (0, 0)),
            pl.BlockSpec((1, C), lambda i: (0, 0)),
        ],
        out_specs=pl.BlockSpec((1024, CW), lambda i: (i, 0)),
        out_shape=jax.ShapeDtypeStruct((NP, CW), jnp.int32),
    )(feats_pad, Wt, b)


def _tree_sum(vs):
    while len(vs) > 1:
        nxt = [vs[i] + vs[i + 1] for i in range(0, len(vs) - 1, 2)]
        if len(vs) % 2:
            nxt.append(vs[-1])
        vs = nxt
    return vs[0]


def _sc_attend_body(f_hbm, ni_hbm, pts_hbm, out_hbm,
                    idx_r, nbrf_r, q_r, out_r, pts_v, accf_v, nidx_s, sems):
    wid = lax.axis_index("s") * 2 + lax.axis_index("c")
    base = wid * PW
    lane_ids = lax.iota(jnp.int32, LANES)
    row_base = lane_ids * LANES
    sem_ni, sem_g, sem_q, sem_o = sems

    def ni_copy(c, b):
        return pltpu.make_async_copy(
            ni_hbm.at[pl.ds((base + c * CH) * K, CH * K)], idx_r[b], sem_ni[b])

    def g_copy(b):
        return pltpu.make_async_copy(f_hbm.at[idx_r[b]], nbrf_r[b], sem_g[b])

    def q_copy(c, b):
        return pltpu.make_async_copy(
            f_hbm.at[pl.ds(base + c * CH, CH)], q_r[b], sem_q[b])

    def o_copy(c, b):
        return pltpu.make_async_copy(
            out_r[b], out_hbm.at[pl.ds(base + c * CH, CH)], sem_o[b])

    for c in range(NBUF):
        ni_copy(c, c).start()
    pltpu.sync_copy(pts_hbm, pts_v)
    for c in range(NBUF - 1):
        ni_copy(c, c).wait()
        g_copy(c).start()
        q_copy(c, c).start()

    def compute_chunk(nbrX, qX, outX):
        def point_body(p, _):
            qs = [plsc.bitcast(qX[p, pl.ds(c * LANES, LANES)], jnp.bfloat16)
                  for c in range(C // 32)]
            for k in range(K):
                accs = [None] * 4
                for c in range(C // 32):
                    pr = qs[c] * plsc.bitcast(
                        nbrX[p * K + k, pl.ds(c * LANES, LANES)], jnp.bfloat16)
                    u0, u1 = plsc.unpack(pr, format=plsc.PackFormat.INTERLEAVED)
                    i0 = 2 * (c % 2)
                    accs[i0] = u0 if accs[i0] is None else accs[i0] + u0
                    accs[i0 + 1] = u1 if accs[i0 + 1] is None else accs[i0 + 1] + u1
                accf_v[pl.ds(k * LANES, LANES)] = (
                    (accs[0] + accs[1]) + (accs[2] + accs[3]))
            dots = _tree_sum(
                [plsc.load_gather(accf_v, [row_base + l]) for l in range(LANES)])
            e = jnp.exp(dots * (1.0 / 16.0))
            nidx = nidx_s[pl.ds(p * K, LANES)] * PTS_W
            px = plsc.load_gather(pts_v, [nidx])
            py = plsc.load_gather(pts_v, [nidx + 1])
            pz = plsc.load_gather(pts_v, [nidx + 2])
            accf_v[pl.ds(0, LANES)] = e * px
            accf_v[pl.ds(LANES, LANES)] = e * py
            accf_v[pl.ds(2 * LANES, LANES)] = e * pz
            accf_v[pl.ds(3 * LANES, LANES)] = e
            t = _tree_sum(
                [plsc.load_gather(accf_v, [row_base + l]) for l in range(LANES)])
            outX[p, :] = t / t[3]
            return 0

        lax.fori_loop(0, CH, point_body, 0)

    def step_body(s, _):
        for b in range(NBUF):
            g = NBUF * s + b
            b3 = (b + NBUF - 1) % NBUF
            g3 = jnp.minimum(g + 3, LAST)
            g4 = jnp.minimum(g + 4, LAST)
            ni_copy(g3, b3).wait()
            g_copy(b3).start()
            q_copy(g3, b3).start()
            g_copy(b).wait()
            q_copy(g, b).wait()
            for j in range(CH):
                nidx_s[pl.ds(j * LANES, LANES)] = idx_r[b][pl.ds(j * LANES, LANES)]
            ni_copy(g4, b).start()

            @pl.when(g >= NBUF)
            def _():
                o_copy(g - NBUF, b).wait()

            compute_chunk(nbrf_r[b], q_r[b], out_r[b])
            o_copy(g, b).start()
        return 0

    lax.fori_loop(0, NCHUNK // NBUF, step_body, 0)

    ni_copy(LAST, 3).wait()
    for b in range(NBUF - 1):
        g_copy(b).wait()
        q_copy(LAST, b).wait()
    for b in range(NBUF):
        o_copy(NCHUNK - NBUF + b, b).wait()


def _sc_attend_entry(f_hbm, ni_hbm, pts_hbm, out_hbm,
                     i0, i1, i2, i3, n0, n1, n2, n3, q0, q1, q2, q3,
                     o0, o1, o2, o3, pts_v, accf_v, nidx_s,
                     sni0, sni1, sni2, sni3, sg0, sg1, sg2, sg3,
                     sq0, sq1, sq2, sq3, so0, so1, so2, so3):
    _sc_attend_body(
        f_hbm, ni_hbm, pts_hbm, out_hbm,
        [i0, i1, i2, i3], [n0, n1, n2, n3], [q0, q1, q2, q3],
        [o0, o1, o2, o3], pts_v, accf_v, nidx_s,
        ([sni0, sni1, sni2, sni3], [sg0, sg1, sg2, sg3],
         [sq0, sq1, sq2, sq3], [so0, so1, so2, so3]))


_sc_attend = functools.partial(
    pl.kernel,
    mesh=plsc.VectorSubcoreMesh(core_axis_name="c", subcore_axis_name="s"),
    compiler_params=pltpu.CompilerParams(needs_layout_passes=False),
    out_type=jax.ShapeDtypeStruct((NP, LANES), jnp.float32),
    scratch_types=(
        [pltpu.VMEM((CH * K,), jnp.int32)] * 4
        + [pltpu.VMEM((CH * K, CW), jnp.int32)] * 4
        + [pltpu.VMEM((CH, CW), jnp.int32)] * 4
        + [pltpu.VMEM((CH, LANES), jnp.float32)] * 4
        + [pltpu.VMEM((N * PTS_W,), jnp.float32),
           pltpu.VMEM((K * LANES,), jnp.float32),
           pltpu.VMEM((CH * K,), jnp.int32)]
        + [pltpu.SemaphoreType.DMA] * 16
    ),
)(_sc_attend_entry)


@jax.jit
def kernel(points, feats, neighbor_indices, W, b):
    ni = neighbor_indices.astype(jnp.int32)
    own = jnp.broadcast_to(jnp.arange(N, dtype=jnp.int32)[:, None], (N, K))
    ni = jnp.where(ni < N, ni, own)
    ni_flat = jnp.pad(ni.reshape(-1), (0, (NP - N) * K))

    feats_pad = jnp.pad(feats, ((0, NP - N), (0, 0)))
    f = _linear(feats_pad, W.T, b[None, :])

    pts_pad = jnp.pad(points, ((0, 0), (0, PTS_W - 3))).reshape(-1)
    out = _sc_attend(f, ni_flat, pts_pad)
    return out[:N, :3]


# speedup vs baseline: 1.4437x; 1.0196x over previous
import functools

import jax
import jax.numpy as jnp
from jax import lax
from jax.experimental import pallas as pl
from jax.experimental.pallas import tpu as pltpu
from jax.experimental.pallas import tpu_sc as plsc

N = 10000
K = 16
C = 256
CW = C // 2
LANES = 16
PTS_W = 4

NW = 32
NP = 10240
PW = NP // NW
CH = 8
NCHUNK = PW // CH
LAST = NCHUNK - 1
NBUF = 4
P0 = 416
P1 = 224


def _mm_body(x_ref, wt_ref, b_ref, o_ref):
    acc = (
        jnp.dot(x_ref[...], wt_ref[...], preferred_element_type=jnp.float32)
        + b_ref[...]
    ).astype(jnp.bfloat16)
    lo = lax.bitcast_convert_type(acc[:, :CW], jnp.uint16).astype(jnp.uint32)
    hi = lax.bitcast_convert_type(acc[:, CW:], jnp.uint16).astype(jnp.uint32)
    o_ref[...] = lax.bitcast_convert_type(lo | (hi << 16), jnp.int32)


def _linear(feats_pad, Wt, b):
    grid = NP // 1024
    return pl.pallas_call(
        _mm_body,
        grid=(grid,),
        in_specs=[
            pl.BlockSpec((1024, C), lambda i: (i, 0)),
            pl.BlockSpec((C, C), lambda i: (0, 0)),
            pl.BlockSpec((1, C), lambda i: (0, 0)),
        ],
        out_specs=pl.BlockSpec((1024, CW), lambda i: (i, 0)),
        out_shape=jax.ShapeDtypeStruct((NP, CW), jnp.int32),
    )(feats_pad, Wt, b)


def _tree_sum(vs):
    while len(vs) > 1:
        nxt = [vs[i] + vs[i + 1] for i in range(0, len(vs) - 1, 2)]
        if len(vs) % 2:
            nxt.append(vs[-1])
        vs = nxt
    return vs[0]


def _sc_attend_body(f_hbm, ni_hbm, pts_hbm, out_hbm,
                    idx_r, nbrf_r, q_r, out_r, pts_v, accf_v, nidx_s, sems):
    lane_ids = lax.iota(jnp.int32, LANES)
    row_base = lane_ids * LANES
    sem_ni, sem_g, sem_q, sem_o = sems
    pltpu.sync_copy(pts_hbm, pts_v)

    def compute_chunk(nbrX, qX, outX):
        def point_body(p, _):
            qs = [plsc.bitcast(qX[p, pl.ds(c * LANES, LANES)], jnp.bfloat16)
                  for c in range(C // 32)]
            for k in range(K):
                accs = [None] * 4
                for c in range(C // 32):
                    pr = qs[c] * plsc.bitcast(
                        nbrX[p * K + k, pl.ds(c * LANES, LANES)], jnp.bfloat16)
                    u0, u1 = plsc.unpack(pr, format=plsc.PackFormat.INTERLEAVED)
                    i0 = 2 * (c % 2)
                    accs[i0] = u0 if accs[i0] is None else accs[i0] + u0
                    accs[i0 + 1] = u1 if accs[i0 + 1] is None else accs[i0 + 1] + u1
                accf_v[pl.ds(k * LANES, LANES)] = (
                    (accs[0] + accs[1]) + (accs[2] + accs[3]))
            dots = _tree_sum(
                [plsc.load_gather(accf_v, [row_base + l]) for l in range(LANES)])
            e = jnp.exp(dots * (1.0 / 16.0))
            nidx = nidx_s[pl.ds(p * K, LANES)] * PTS_W
            px = plsc.load_gather(pts_v, [nidx])
            py = plsc.load_gather(pts_v, [nidx + 1])
            pz = plsc.load_gather(pts_v, [nidx + 2])
            accf_v[pl.ds(0, LANES)] = e * px
            accf_v[pl.ds(LANES, LANES)] = e * py
            accf_v[pl.ds(2 * LANES, LANES)] = e * pz
            accf_v[pl.ds(3 * LANES, LANES)] = e
            t = _tree_sum(
                [plsc.load_gather(accf_v, [row_base + l]) for l in range(LANES)])
            outX[p, :] = t / t[3]
            return 0

        lax.fori_loop(0, CH, point_body, 0)

    def pipeline(base, nchunk):
        last = nchunk - 1

        def ni_copy(c, b):
            return pltpu.make_async_copy(
                ni_hbm.at[pl.ds((base + c * CH) * K, CH * K)],
                idx_r[b], sem_ni[b])

        def g_copy(b):
            return pltpu.make_async_copy(
                f_hbm.at[idx_r[b]], nbrf_r[b], sem_g[b])

        def q_copy(c, b):
            return pltpu.make_async_copy(
                f_hbm.at[pl.ds(base + c * CH, CH)], q_r[b], sem_q[b])

        def o_copy(c, b):
            return pltpu.make_async_copy(
                out_r[b], out_hbm.at[pl.ds(base + c * CH, CH)], sem_o[b])

        for c in range(NBUF):
            ni_copy(c, c).start()
        for c in range(NBUF - 1):
            ni_copy(c, c).wait()
            g_copy(c).start()
            q_copy(c, c).start()

        def step_body(s, _):
            for b in range(NBUF):
                g = NBUF * s + b
                b3 = (b + NBUF - 1) % NBUF
                g3 = jnp.minimum(g + 3, last)
                g4 = jnp.minimum(g + 4, last)
                ni_copy(g3, b3).wait()
                g_copy(b3).start()
                q_copy(g3, b3).start()
                g_copy(b).wait()
                q_copy(g, b).wait()
                for j in range(CH):
                    nidx_s[pl.ds(j * LANES, LANES)] = (
                        idx_r[b][pl.ds(j * LANES, LANES)])
                ni_copy(g4, b).start()

                @pl.when(g >= NBUF)
                def _():
                    o_copy(g - NBUF, b).wait()

                compute_chunk(nbrf_r[b], q_r[b], out_r[b])
                o_copy(g, b).start()
            return 0

        lax.fori_loop(0, nchunk // NBUF, step_body, 0)

        ni_copy(last, 3).wait()
        for b in range(NBUF - 1):
            g_copy(b).wait()
            q_copy(last, b).wait()
        for b in range(NBUF):
            o_copy(nchunk - NBUF + b, b).wait()

    sid = lax.axis_index("s")

    @pl.when(lax.axis_index("c") == 0)
    def _():
        pipeline(sid * (P0 + P1), P0 // CH)

    @pl.when(lax.axis_index("c") == 1)
    def _():
        pipeline(sid * (P0 + P1) + P0, P1 // CH)


def _sc_attend_entry(f_hbm, ni_hbm, pts_hbm, out_hbm,
                     i0, i1, i2, i3, n0, n1, n2, n3, q0, q1, q2, q3,
                     o0, o1, o2, o3, pts_v, accf_v, nidx_s,
                     sni0, sni1, sni2, sni3, sg0, sg1, sg2, sg3,
                     sq0, sq1, sq2, sq3, so0, so1, so2, so3):
    _sc_attend_body(
        f_hbm, ni_hbm, pts_hbm, out_hbm,
        [i0, i1, i2, i3], [n0, n1, n2, n3], [q0, q1, q2, q3],
        [o0, o1, o2, o3], pts_v, accf_v, nidx_s,
        ([sni0, sni1, sni2, sni3], [sg0, sg1, sg2, sg3],
         [sq0, sq1, sq2, sq3], [so0, so1, so2, so3]))


_sc_attend = functools.partial(
    pl.kernel,
    mesh=plsc.VectorSubcoreMesh(core_axis_name="c", subcore_axis_name="s"),
    compiler_params=pltpu.CompilerParams(needs_layout_passes=False),
    out_type=jax.ShapeDtypeStruct((NP, LANES), jnp.float32),
    scratch_types=(
        [pltpu.VMEM((CH * K,), jnp.int32)] * 4
        + [pltpu.VMEM((CH * K, CW), jnp.int32)] * 4
        + [pltpu.VMEM((CH, CW), jnp.int32)] * 4
        + [pltpu.VMEM((CH, LANES), jnp.float32)] * 4
        + [pltpu.VMEM((N * PTS_W,), jnp.float32),
           pltpu.VMEM((K * LANES,), jnp.float32),
           pltpu.VMEM((CH * K,), jnp.int32)]
        + [pltpu.SemaphoreType.DMA] * 16
    ),
)(_sc_attend_entry)


@jax.jit
def kernel(points, feats, neighbor_indices, W, b):
    ni = neighbor_indices.astype(jnp.int32)
    own = jnp.broadcast_to(jnp.arange(N, dtype=jnp.int32)[:, None], (N, K))
    ni = jnp.where(ni < N, ni, own)
    ni_flat = jnp.pad(ni.reshape(-1), (0, (NP - N) * K))

    feats_pad = jnp.pad(feats, ((0, NP - N), (0, 0)))
    f = _linear(feats_pad, W.T, b[None, :])

    pts_pad = jnp.pad(points, ((0, 0), (0, PTS_W - 3))).reshape(-1)
    out = _sc_attend(f, ni_flat, pts_pad)
    return out[:N, :3]


# speedup vs baseline: 1.4864x; 1.0296x over previous
import functools

import jax
import jax.numpy as jnp
from jax import lax
from jax.experimental import pallas as pl
from jax.experimental.pallas import tpu as pltpu
from jax.experimental.pallas import tpu_sc as plsc

N = 10000
K = 16
C = 256
CW = C // 2
LANES = 16
PTS_W = 4

NW = 32
NP = 10240
PW = NP // NW
CH = 8
NCHUNK = PW // CH
LAST = NCHUNK - 1
NBUF = 4
P0 = 416
P1 = 224


def _mm_body(x_ref, w_ref, b_ref, o_ref):
    acc = (
        lax.dot_general(
            x_ref[...], w_ref[...], (((1,), (1,)), ((), ())),
            preferred_element_type=jnp.float32)
        + b_ref[...]
    ).astype(jnp.bfloat16)
    lo = lax.bitcast_convert_type(acc[:, :CW], jnp.uint16).astype(jnp.uint32)
    hi = lax.bitcast_convert_type(acc[:, CW:], jnp.uint16).astype(jnp.uint32)
    o_ref[...] = lax.bitcast_convert_type(lo | (hi << 16), jnp.int32)


def _linear(feats, W, b):
    grid = NP // 1024
    return pl.pallas_call(
        _mm_body,
        grid=(grid,),
        in_specs=[
            pl.BlockSpec((1024, C), lambda i: (i, 0)),
            pl.BlockSpec((C, C), lambda i: (0, 0)),
            pl.BlockSpec((1, C), lambda i: (0, 0)),
        ],
        out_specs=pl.BlockSpec((1024, CW), lambda i: (i, 0)),
        out_shape=jax.ShapeDtypeStruct((NP, CW), jnp.int32),
    )(feats, W, b)


def _tree_sum(vs):
    while len(vs) > 1:
        nxt = [vs[i] + vs[i + 1] for i in range(0, len(vs) - 1, 2)]
        if len(vs) % 2:
            nxt.append(vs[-1])
        vs = nxt
    return vs[0]


def _sc_attend_body(f_hbm, ni_hbm, pts_hbm, out_hbm,
                    idx_r, nbrf_r, q_r, out_r, pts_v, accf_v, nidx_s, sems):
    lane_ids = lax.iota(jnp.int32, LANES)
    row_base = lane_ids * LANES
    sem_ni, sem_g, sem_q, sem_o = sems
    pltpu.sync_copy(pts_hbm, pts_v)

    def compute_chunk(nbrX, qX, outX):
        def point_body(p, _):
            qs = [plsc.bitcast(qX[p, pl.ds(c * LANES, LANES)], jnp.bfloat16)
                  for c in range(C // 32)]
            for k in range(K):
                accs = [None] * 4
                for c in range(C // 32):
                    pr = qs[c] * plsc.bitcast(
                        nbrX[p * K + k, pl.ds(c * LANES, LANES)], jnp.bfloat16)
                    u0, u1 = plsc.unpack(pr, format=plsc.PackFormat.INTERLEAVED)
                    i0 = 2 * (c % 2)
                    accs[i0] = u0 if accs[i0] is None else accs[i0] + u0
                    accs[i0 + 1] = u1 if accs[i0 + 1] is None else accs[i0 + 1] + u1
                accf_v[pl.ds(k * LANES, LANES)] = (
                    (accs[0] + accs[1]) + (accs[2] + accs[3]))
            dots = _tree_sum(
                [plsc.load_gather(accf_v, [row_base + l]) for l in range(LANES)])
            e = jnp.exp(dots * (1.0 / 16.0))
            ni_k = nidx_s[pl.ds(p * K, LANES)]
            nidx = ni_k * 2 + ni_k
            px = plsc.load_gather(pts_v, [nidx])
            py = plsc.load_gather(pts_v, [nidx + 1])
            pz = plsc.load_gather(pts_v, [nidx + 2])
            accf_v[pl.ds(0, LANES)] = e * px
            accf_v[pl.ds(LANES, LANES)] = e * py
            accf_v[pl.ds(2 * LANES, LANES)] = e * pz
            accf_v[pl.ds(3 * LANES, LANES)] = e
            t = _tree_sum(
                [plsc.load_gather(accf_v, [row_base + l]) for l in range(LANES)])
            outX[p, :] = t / t[3]
            return 0

        lax.fori_loop(0, CH, point_body, 0)

    def pipeline(base, nchunk):
        last = nchunk - 1

        def ni_copy(c, b):
            return pltpu.make_async_copy(
                ni_hbm.at[pl.ds((base + c * CH) * K, CH * K)],
                idx_r[b], sem_ni[b])

        def g_copy(b):
            return pltpu.make_async_copy(
                f_hbm.at[idx_r[b]], nbrf_r[b], sem_g[b])

        def q_copy(c, b):
            return pltpu.make_async_copy(
                f_hbm.at[pl.ds(base + c * CH, CH)], q_r[b], sem_q[b])

        def o_copy(c, b):
            return pltpu.make_async_copy(
                out_r[b], out_hbm.at[pl.ds(base + c * CH, CH)], sem_o[b])

        for c in range(NBUF):
            ni_copy(c, c).start()
        for c in range(NBUF - 1):
            ni_copy(c, c).wait()
            g_copy(c).start()
            q_copy(c, c).start()

        def step_body(s, _):
            for b in range(NBUF):
                g = NBUF * s + b
                b3 = (b + NBUF - 1) % NBUF
                g3 = jnp.minimum(g + 3, last)
                g4 = jnp.minimum(g + 4, last)
                ni_copy(g3, b3).wait()
                g_copy(b3).start()
                q_copy(g3, b3).start()
                g_copy(b).wait()
                q_copy(g, b).wait()
                for j in range(CH):
                    nidx_s[pl.ds(j * LANES, LANES)] = (
                        idx_r[b][pl.ds(j * LANES, LANES)])
                ni_copy(g4, b).start()

                @pl.when(g >= NBUF)
                def _():
                    o_copy(g - NBUF, b).wait()

                compute_chunk(nbrf_r[b], q_r[b], out_r[b])
                o_copy(g, b).start()
            return 0

        lax.fori_loop(0, nchunk // NBUF, step_body, 0)

        ni_copy(last, 3).wait()
        for b in range(NBUF - 1):
            g_copy(b).wait()
            q_copy(last, b).wait()
        for b in range(NBUF):
            o_copy(nchunk - NBUF + b, b).wait()

    sid = lax.axis_index("s")

    @pl.when(lax.axis_index("c") == 0)
    def _():
        pipeline(sid * (P0 + P1), P0 // CH)

    @pl.when(lax.axis_index("c") == 1)
    def _():
        pipeline(sid * (P0 + P1) + P0, P1 // CH)


def _sc_attend_entry(f_hbm, ni_hbm, pts_hbm, out_hbm,
                     i0, i1, i2, i3, n0, n1, n2, n3, q0, q1, q2, q3,
                     o0, o1, o2, o3, pts_v, accf_v, nidx_s,
                     sni0, sni1, sni2, sni3, sg0, sg1, sg2, sg3,
                     sq0, sq1, sq2, sq3, so0, so1, so2, so3):
    _sc_attend_body(
        f_hbm, ni_hbm, pts_hbm, out_hbm,
        [i0, i1, i2, i3], [n0, n1, n2, n3], [q0, q1, q2, q3],
        [o0, o1, o2, o3], pts_v, accf_v, nidx_s,
        ([sni0, sni1, sni2, sni3], [sg0, sg1, sg2, sg3],
         [sq0, sq1, sq2, sq3], [so0, so1, so2, so3]))


_sc_attend = functools.partial(
    pl.kernel,
    mesh=plsc.VectorSubcoreMesh(core_axis_name="c", subcore_axis_name="s"),
    compiler_params=pltpu.CompilerParams(needs_layout_passes=False),
    out_type=jax.ShapeDtypeStruct((NP, LANES), jnp.float32),
    scratch_types=(
        [pltpu.VMEM((CH * K,), jnp.int32)] * 4
        + [pltpu.VMEM((CH * K, CW), jnp.int32)] * 4
        + [pltpu.VMEM((CH, CW), jnp.int32)] * 4
        + [pltpu.VMEM((CH, LANES), jnp.float32)] * 4
        + [pltpu.VMEM((N * 3,), jnp.float32),
           pltpu.VMEM((K * LANES,), jnp.float32),
           pltpu.VMEM((CH * K,), jnp.int32)]
        + [pltpu.SemaphoreType.DMA] * 16
    ),
)(_sc_attend_entry)


@jax.jit
def kernel(points, feats, neighbor_indices, W, b):
    ni = neighbor_indices.astype(jnp.int32)
    own = jnp.broadcast_to(jnp.arange(N, dtype=jnp.int32)[:, None], (N, K))
    ni = jnp.where(ni < N, ni, own)
    ni_flat = jnp.pad(ni.reshape(-1), (0, (NP - N) * K))

    f = _linear(feats, W, b[None, :])

    out = _sc_attend(f, ni_flat, points.reshape(-1))
    return out[:N, :3]


# speedup vs baseline: 1.5035x; 1.0115x over previous
import functools

import jax
import jax.numpy as jnp
from jax import lax
from jax.experimental import pallas as pl
from jax.experimental.pallas import tpu as pltpu
from jax.experimental.pallas import tpu_sc as plsc

N = 10000
K = 16
C = 256
CW = C // 2
LANES = 16
PTS_W = 4

NW = 32
NP = 10240
PW = NP // NW
CH = 8
NCHUNK = PW // CH
LAST = NCHUNK - 1
NBUF = 4
P0 = 320
P1 = 320


def _mm_body(x_ref, w_ref, b_ref, o_ref):
    acc = (
        lax.dot_general(
            x_ref[...], w_ref[...], (((1,), (1,)), ((), ())),
            preferred_element_type=jnp.float32)
        + b_ref[...]
    ).astype(jnp.bfloat16)
    lo = lax.bitcast_convert_type(acc[:, :CW], jnp.uint16).astype(jnp.uint32)
    hi = lax.bitcast_convert_type(acc[:, CW:], jnp.uint16).astype(jnp.uint32)
    o_ref[...] = lax.bitcast_convert_type(lo | (hi << 16), jnp.int32)


def _linear(feats, W, b):
    grid = NP // 1024
    return pl.pallas_call(
        _mm_body,
        grid=(grid,),
        in_specs=[
            pl.BlockSpec((1024, C), lambda i: (i, 0)),
            pl.BlockSpec((C, C), lambda i: (0, 0)),
            pl.BlockSpec((1, C), lambda i: (0, 0)),
        ],
        out_specs=pl.BlockSpec((1024, CW), lambda i: (i, 0)),
        out_shape=jax.ShapeDtypeStruct((NP, CW), jnp.int32),
    )(feats, W, b)


def _tree_sum(vs):
    while len(vs) > 1:
        nxt = [vs[i] + vs[i + 1] for i in range(0, len(vs) - 1, 2)]
        if len(vs) % 2:
            nxt.append(vs[-1])
        vs = nxt
    return vs[0]


def _sc_attend_body(f_hbm, ni_hbm, pts_hbm, out_hbm,
                    idx_r, nbrf_r, q_r, out_r, pts_v, accf_v, nidx_s, sems):
    lane_ids = lax.iota(jnp.int32, LANES)
    row_base = lane_ids * LANES
    sem_ni, sem_g, sem_q, sem_o = sems
    pltpu.sync_copy(pts_hbm, pts_v)

    def compute_chunk(nbrX, qX, outX):
        def point_body(p, _):
            qs = [plsc.bitcast(qX[p, pl.ds(c * LANES, LANES)], jnp.bfloat16)
                  for c in range(C // 32)]
            for k in range(K):
                accs = [None] * 4
                for c in range(C // 32):
                    pr = qs[c] * plsc.bitcast(
                        nbrX[p * K + k, pl.ds(c * LANES, LANES)], jnp.bfloat16)
                    u0, u1 = plsc.unpack(pr, format=plsc.PackFormat.INTERLEAVED)
                    i0 = 2 * (c % 2)
                    accs[i0] = u0 if accs[i0] is None else accs[i0] + u0
                    accs[i0 + 1] = u1 if accs[i0 + 1] is None else accs[i0 + 1] + u1
                accf_v[pl.ds(k * LANES, LANES)] = (
                    (accs[0] + accs[1]) + (accs[2] + accs[3]))
            dots = _tree_sum(
                [plsc.load_gather(accf_v, [row_base + l]) for l in range(LANES)])
            e = jnp.exp(dots * (1.0 / 16.0))
            ni_k = nidx_s[pl.ds(p * K, LANES)]
            nidx = ni_k * 2 + ni_k
            px = plsc.load_gather(pts_v, [nidx])
            py = plsc.load_gather(pts_v, [nidx + 1])
            pz = plsc.load_gather(pts_v, [nidx + 2])
            accf_v[pl.ds(0, LANES)] = e * px
            accf_v[pl.ds(LANES, LANES)] = e * py
            accf_v[pl.ds(2 * LANES, LANES)] = e * pz
            accf_v[pl.ds(3 * LANES, LANES)] = e
            t = _tree_sum(
                [plsc.load_gather(accf_v, [row_base + l]) for l in range(LANES)])
            outX[p, :] = t / t[3]
            return 0

        lax.fori_loop(0, CH, point_body, 0)

    def pipeline(base, nchunk):
        last = nchunk - 1

        def ni_copy(c, b):
            return pltpu.make_async_copy(
                ni_hbm.at[pl.ds((base + c * CH) * K, CH * K)],
                idx_r[b], sem_ni[b])

        def g_copy(b):
            return pltpu.make_async_copy(
                f_hbm.at[idx_r[b]], nbrf_r[b], sem_g[b])

        def q_copy(c, b):
            return pltpu.make_async_copy(
                f_hbm.at[pl.ds(base + c * CH, CH)], q_r[b], sem_q[b])

        def o_copy(c, b):
            return pltpu.make_async_copy(
                out_r[b], out_hbm.at[pl.ds(base + c * CH, CH)], sem_o[b])

        for c in range(NBUF):
            ni_copy(c, c).start()
        for c in range(NBUF - 1):
            ni_copy(c, c).wait()
            g_copy(c).start()
            q_copy(c, c).start()

        def step_body(s, _):
            for b in range(NBUF):
                g = NBUF * s + b
                b3 = (b + NBUF - 1) % NBUF
                g3 = jnp.minimum(g + 3, last)
                g4 = jnp.minimum(g + 4, last)
                ni_copy(g3, b3).wait()
                g_copy(b3).start()
                q_copy(g3, b3).start()
                g_copy(b).wait()
                q_copy(g, b).wait()
                for j in range(CH):
                    nidx_s[pl.ds(j * LANES, LANES)] = (
                        idx_r[b][pl.ds(j * LANES, LANES)])
                ni_copy(g4, b).start()

                @pl.when(g >= NBUF)
                def _():
                    o_copy(g - NBUF, b).wait()

                compute_chunk(nbrf_r[b], q_r[b], out_r[b])
                o_copy(g, b).start()
            return 0

        lax.fori_loop(0, nchunk // NBUF, step_body, 0)

        ni_copy(last, 3).wait()
        for b in range(NBUF - 1):
            g_copy(b).wait()
            q_copy(last, b).wait()
        for b in range(NBUF):
            o_copy(nchunk - NBUF + b, b).wait()

    sid = lax.axis_index("s")

    @pl.when(lax.axis_index("c") == 0)
    def _():
        pipeline(sid * (P0 + P1), P0 // CH)

    @pl.when(lax.axis_index("c") == 1)
    def _():
        pipeline(sid * (P0 + P1) + P0, P1 // CH)


def _sc_attend_entry(f_hbm, ni_hbm, pts_hbm, out_hbm,
                     i0, i1, i2, i3, n0, n1, n2, n3, q0, q1, q2, q3,
                     o0, o1, o2, o3, pts_v, accf_v, nidx_s,
                     sni0, sni1, sni2, sni3, sg0, sg1, sg2, sg3,
                     sq0, sq1, sq2, sq3, so0, so1, so2, so3):
    _sc_attend_body(
        f_hbm, ni_hbm, pts_hbm, out_hbm,
        [i0, i1, i2, i3], [n0, n1, n2, n3], [q0, q1, q2, q3],
        [o0, o1, o2, o3], pts_v, accf_v, nidx_s,
        ([sni0, sni1, sni2, sni3], [sg0, sg1, sg2, sg3],
         [sq0, sq1, sq2, sq3], [so0, so1, so2, so3]))


_sc_attend = functools.partial(
    pl.kernel,
    mesh=plsc.VectorSubcoreMesh(core_axis_name="c", subcore_axis_name="s"),
    compiler_params=pltpu.CompilerParams(needs_layout_passes=False),
    out_type=jax.ShapeDtypeStruct((NP, LANES), jnp.float32),
    scratch_types=(
        [pltpu.VMEM((CH * K,), jnp.int32)] * 4
        + [pltpu.VMEM((CH * K, CW), jnp.int32)] * 4
        + [pltpu.VMEM((CH, CW), jnp.int32)] * 4
        + [pltpu.VMEM((CH, LANES), jnp.float32)] * 4
        + [pltpu.VMEM((N * 3,), jnp.float32),
           pltpu.VMEM((K * LANES,), jnp.float32),
           pltpu.VMEM((CH * K,), jnp.int32)]
        + [pltpu.SemaphoreType.DMA] * 16
    ),
)(_sc_attend_entry)


@jax.jit
def kernel(points, feats, neighbor_indices, W, b):
    ni = neighbor_indices.astype(jnp.int32)
    own = jnp.broadcast_to(jnp.arange(N, dtype=jnp.int32)[:, None], (N, K))
    ni = jnp.where(ni < N, ni, own)
    ni_flat = jnp.pad(ni.reshape(-1), (0, (NP - N) * K))

    f = _linear(feats, W, b[None, :])

    out = _sc_attend(f, ni_flat, points.reshape(-1))
    return out[:N, :3]
